# baseline (speedup 1.0000x reference)
import functools

import jax
import jax.numpy as jnp
from jax import lax
from jax.experimental import pallas as pl
from jax.experimental.pallas import tpu as pltpu
from jax.experimental.pallas import tpu_sc as plsc

_N = 50000
_E = 1600000
_R = 8
_D = 32

_NC = 2
_NS = 16
_NW = _NC * _NS
_EW = _E // _NW
_K = 80
_CPS = 25
_SBE = _K * _CPS
_NSB = _EW // _SBE
_RPT = _N // _NS
_NR = _N * _R
_RPTC = _NR // _NS
_CROWS = _EW // _K
_NCH = _EW // _K

_BN = 2000


def _mesh():
    return plsc.VectorSubcoreMesh(core_axis_name="c", subcore_axis_name="s")


def _cnt_body(seg2_hbm, zc_hbm, out_hbm, seg_v, ones_v, acc_sh):
    c = lax.axis_index("c")
    s = lax.axis_index("s")
    wid = c * _NS + s
    pltpu.sync_copy(zc_hbm, acc_sh.at[pl.ds(s * _RPTC, _RPTC)])
    for t in range(_K // 16):
        ones_v[pl.ds(t * 16, 16)] = jnp.ones((16,), jnp.float32)
    plsc.subcore_barrier()
    cbase = wid * _CROWS

    def sb_body(j, carry):
        pltpu.sync_copy(seg2_hbm.at[pl.ds(cbase + j * _CPS, _CPS)], seg_v)

        def ch_body(i, c2):
            pltpu.sync_copy(ones_v, acc_sh.at[seg_v.at[i]], add=True)
            return c2

        return lax.fori_loop(0, _CPS, ch_body, carry)

    lax.fori_loop(0, _NSB, sb_body, 0)
    plsc.subcore_barrier()
    pltpu.sync_copy(acc_sh.at[pl.ds(s * _RPTC, _RPTC)],
                    out_hbm.at[pl.ds(c * _NR + s * _RPTC, _RPTC)])


def _sc_counts(seg2, zc):
    f = pl.kernel(
        _cnt_body,
        out_type=jax.ShapeDtypeStruct((_NC * _NR,), jnp.float32),
        mesh=_mesh(),
        compiler_params=pltpu.CompilerParams(use_tc_tiling_on_sc=False, needs_layout_passes=False),
        scratch_types=[
            pltpu.VMEM((_CPS, _K), jnp.int32),
            pltpu.VMEM((_K,), jnp.float32),
            pltpu.VMEM_SHARED((_NR,), jnp.float32),
        ],
    )
    return f(seg2, zc)


def _scl_body(seg_hbm, invc_hbm, out_hbm, seg0, seg1, scl0, scl1, g0, g1):
    seg = [seg0, seg1]
    scl = [scl0, scl1]
    gsem = [g0, g1]
    c_ax = lax.axis_index("c")
    s_ax = lax.axis_index("s")
    wid = c_ax * _NS + s_ax
    ebase = wid * _EW

    def fire(ps, r):
        pltpu.async_copy(invc_hbm.at[seg[ps].at[pl.ds(r * _K, _K)]],
                         scl[ps].at[pl.ds(r * _K, _K)], gsem[ps])

    def drain(ps):
        pltpu.make_async_copy(invc_hbm.at[seg[ps].at[pl.ds(0, _K)]],
                              scl[ps].at[pl.ds(0, _K)], gsem[ps]).wait()

    def sb_loop(jj, carry):
        for u in range(2):
            j = jj * 2 + u
            pltpu.sync_copy(seg_hbm.at[pl.ds(ebase + j * _SBE, _SBE)], seg[u])
            fire(u, 0)

            def ch(i, c2):
                @pl.when(i + 1 < _CPS)
                def _():
                    fire(u, i + 1)

                drain(u)
                return c2

            lax.fori_loop(0, _CPS, ch, 0)
            pltpu.sync_copy(scl[u], out_hbm.at[pl.ds(ebase + j * _SBE, _SBE)])
        return carry

    lax.fori_loop(0, _NSB // 2, sb_loop, 0)
    jt = _NSB - 1
    pltpu.sync_copy(seg_hbm.at[pl.ds(ebase + jt * _SBE, _SBE)], seg[0])
    fire(0, 0)

    def cht(i, c2):
        @pl.when(i + 1 < _CPS)
        def _():
            fire(0, i + 1)

        drain(0)
        return c2

    lax.fori_loop(0, _CPS, cht, 0)
    pltpu.sync_copy(scl[0], out_hbm.at[pl.ds(ebase + jt * _SBE, _SBE)])


def _sc_scale(seg, invc):
    f = pl.kernel(
        _scl_body,
        out_type=jax.ShapeDtypeStruct((_E,), jnp.float32),
        mesh=_mesh(),
        compiler_params=pltpu.CompilerParams(use_tc_tiling_on_sc=False, needs_layout_passes=False),
        scratch_types=[
            pltpu.VMEM((_SBE,), jnp.int32),
            pltpu.VMEM((_SBE,), jnp.int32),
            pltpu.VMEM((_SBE,), jnp.float32),
            pltpu.VMEM((_SBE,), jnp.float32),
            pltpu.SemaphoreType.DMA,
            pltpu.SemaphoreType.DMA,
        ],
    )
    return f(seg, invc)


def _agg_body(hw_hbm, edata_hbm, zrows_hbm, out_hbm,
              edata_v, acc_sh, rows0, rows1,
              srows0, srows1, srows2, srows3,
              g0, g1, s0, s1, s2, s3):
    rows = [rows0, rows1]
    srows = [srows0, srows1, srows2, srows3]
    gsem = [g0, g1]
    ssem = [s0, s1, s2, s3]
    c_ax = lax.axis_index("c")
    s_ax = lax.axis_index("s")
    wid = c_ax * _NS + s_ax
    pltpu.sync_copy(zrows_hbm, acc_sh.at[pl.ds(s_ax * _RPT, _RPT)])
    plsc.subcore_barrier()
    cbase = wid * _CROWS

    def lin_load(row0, ps):
        pltpu.sync_copy(edata_hbm.at[pl.ds(cbase + row0, _CPS)],
                        edata_v.at[ps])

    def fire_gather(ps, r, q):
        pltpu.async_copy(hw_hbm.at[edata_v.at[ps, r, 0]], rows[q], gsem[q])

    def drain_gather(q):
        pltpu.make_async_copy(hw_hbm.at[edata_v.at[0, 0, 0]],
                              rows[q], gsem[q]).wait()

    def fire_scatter(ps, r, b):
        pltpu.async_copy(srows[b], acc_sh.at[edata_v.at[ps, r, 1]],
                         ssem[b], add=True)

    def drain_scatter(b):
        pltpu.make_async_copy(srows[b], acc_sh.at[edata_v.at[0, 0, 1]],
                              ssem[b]).wait()

    def compute(ps, r, q, b):
        psv = jnp.full((16,), ps, jnp.int32)
        rv = jnp.full((16,), r, jnp.int32)
        two = jnp.full((16,), 2, jnp.int32)
        bufq = rows[q]
        sbufq = srows[b]
        for k in range(_K):
            kv = jnp.full((16,), k, jnp.int32)
            sc = plsc.bitcast(plsc.load_gather(edata_v, [psv, rv, two, kv]),
                              jnp.float32)
            sbufq[k, pl.ds(0, 16)] = bufq[k, pl.ds(0, 16)] * sc
            sbufq[k, pl.ds(16, 16)] = bufq[k, pl.ds(16, 16)] * sc

    def advance(ps, r):
        wrap = (r + 1) == _CPS
        return (jnp.where(wrap, 1 - ps, ps).astype(jnp.int32),
                jnp.where(wrap, 0, r + 1).astype(jnp.int32))

    lin_load(0, 0)
    fire_gather(0, 0, 0)

    def step(t, carry):
        psA, rA, psB, rB = carry
        for k in range(4):
            c = t * 4 + k
            q = k & 1
            cn = c + 1

            @pl.when(rA == 0)
            def _():
                lin_load(cn, psA)

            @pl.when(cn < _NCH)
            def _():
                fire_gather(psA, rA, 1 - q)

            drain_gather(q)

            @pl.when(c >= 4)
            def _():
                drain_scatter(k)

            compute(psB, rB, q, k)
            fire_scatter(psB, rB, k)
            psA, rA = advance(psA, rA)
            psB, rB = advance(psB, rB)
        return (psA, rA, psB, rB)

    z = jnp.int32(0)
    carry = lax.fori_loop(0, (_NCH - 1) // 4, step,
                          (z, jnp.int32(1), z, z))
    psB, rB = carry[2], carry[3]
    drain_gather(0)
    drain_scatter(0)
    compute(psB, rB, 0, 0)
    fire_scatter(psB, rB, 0)
    drain_scatter(1)
    drain_scatter(2)
    drain_scatter(3)
    drain_scatter(0)
    plsc.subcore_barrier()
    pltpu.sync_copy(acc_sh.at[pl.ds(s_ax * _RPT, _RPT)],
                    out_hbm.at[pl.ds(c_ax * _N + s_ax * _RPT, _RPT)])


def _sc_agg(hw, edata, zrows):
    f = pl.kernel(
        _agg_body,
        out_type=jax.ShapeDtypeStruct((_NC * _N, _D), jnp.float32),
        mesh=_mesh(),
        compiler_params=pltpu.CompilerParams(use_tc_tiling_on_sc=False, needs_layout_passes=False),
        scratch_types=[
            pltpu.VMEM((2, _CPS, 3, _K), jnp.int32),
            pltpu.VMEM_SHARED((_N, _D), jnp.float32),
        ] + [pltpu.VMEM((_K, _D), jnp.float32)] * 6
          + [pltpu.SemaphoreType.DMA] * 6,
    )
    return f(hw, edata, zrows)


def _invc_body(c_ref, o_ref):
    csum = c_ref[0] + c_ref[1]
    o_ref[...] = 1.0 / jnp.maximum(csum, 1.0)


def _tc_invc(counts3):
    return pl.pallas_call(
        _invc_body,
        grid=(1,),
        in_specs=[pl.BlockSpec((2, _NR // 128, 128), lambda i: (0, 0, 0))],
        out_specs=pl.BlockSpec((_NR // 128, 128), lambda i: (0, 0)),
        out_shape=jax.ShapeDtypeStruct((_NR // 128, 128), jnp.float32),
    )(counts3)


def _proj_body(h_ref, basesT_ref, cmat_ref, root_ref, bias_ref, hw_ref, rp_ref):
    h = h_ref[...]
    wcat = jnp.dot(basesT_ref[...], cmat_ref[...],
                   preferred_element_type=jnp.float32)
    hw_ref[...] = jnp.dot(h, wcat, preferred_element_type=jnp.float32)
    rp_ref[...] = (jnp.dot(h, root_ref[...], preferred_element_type=jnp.float32)
                   + bias_ref[...])


def _fused_body(p0_ref, p1_ref, rpp_ref, basesT_ref, cmat_ref, root_ref,
                bias_ref, hw_ref, rp_ref):
    h = jnp.maximum(p0_ref[...] + p1_ref[...] + rpp_ref[...], 0.0)
    wcat = jnp.dot(basesT_ref[...], cmat_ref[...],
                   preferred_element_type=jnp.float32)
    hw_ref[...] = jnp.dot(h, wcat, preferred_element_type=jnp.float32)
    rp_ref[...] = (jnp.dot(h, root_ref[...], preferred_element_type=jnp.float32)
                   + bias_ref[...])


_W_SPECS = [
    pl.BlockSpec((_D, _R * _D), lambda i: (0, 0)),
    pl.BlockSpec((_R * _D, _R * _D), lambda i: (0, 0)),
    pl.BlockSpec((_D, _D), lambda i: (0, 0)),
    pl.BlockSpec((1, _D), lambda i: (0, 0)),
]

_OUT_SPECS = [
    pl.BlockSpec((_BN, _R * _D), lambda i: (i, 0)),
    pl.BlockSpec((_BN, _D), lambda i: (i, 0)),
]

_OUT_SHAPES = [
    jax.ShapeDtypeStruct((_N, _R * _D), jnp.float32),
    jax.ShapeDtypeStruct((_N, _D), jnp.float32),
]


def _tc_proj(h, basesT, cmat, rootl, biasl):
    return pl.pallas_call(
        _proj_body,
        grid=(_N // _BN,),
        in_specs=[pl.BlockSpec((_BN, _D), lambda i: (i, 0))] + _W_SPECS,
        out_specs=_OUT_SPECS,
        out_shape=_OUT_SHAPES,
    )(h, basesT, cmat, rootl, biasl)


def _tc_fused(p0, p1, rpp, basesT, cmat, rootl, biasl):
    nspec = pl.BlockSpec((_BN, _D), lambda i: (i, 0))
    return pl.pallas_call(
        _fused_body,
        grid=(_N // _BN,),
        in_specs=[nspec, nspec, nspec] + _W_SPECS,
        out_specs=_OUT_SPECS,
        out_shape=_OUT_SHAPES,
    )(p0, p1, rpp, basesT, cmat, rootl, biasl)


def _fin_body(p0_ref, p1_ref, rp_ref, o_ref):
    o_ref[...] = p0_ref[...] + p1_ref[...] + rp_ref[...]


def _tc_final(p0, p1, rp):
    spec = pl.BlockSpec((2000, _D), lambda i: (i, 0))
    return pl.pallas_call(
        _fin_body,
        grid=(_N // 2000,),
        in_specs=[spec, spec, spec],
        out_specs=spec,
        out_shape=jax.ShapeDtypeStruct((_N, _D), jnp.float32),
    )(p0, p1, rp)


def kernel(x, edge_index, edge_type, embed, comp, bases, root, bias):
    h = jnp.take(embed, x, axis=0)
    src = edge_index[0]
    dst = edge_index[1]
    et = edge_type
    gidx = src * _R + et
    seg = dst * _R + et
    dst2 = dst.reshape(_E // _K, _K)
    seg2 = seg.reshape(_E // _K, _K)
    zrows = jnp.zeros((_RPT, _D), jnp.float32)
    zc = jnp.zeros((_RPTC,), jnp.float32)
    eye = jnp.eye(_D, dtype=jnp.float32)

    cnts = _sc_counts(seg2, zc)
    invc = _tc_invc(cnts.reshape(2, _NR // 128, 128)).reshape(_NR)
    scale = _sc_scale(seg, invc)
    scale_i = lax.bitcast_convert_type(scale, jnp.int32)
    edata = jnp.stack([gidx.reshape(_E // _K, _K), dst2,
                       scale_i.reshape(_E // _K, _K)], axis=1)

    p0 = p1 = rp = None
    for l in range(3):
        basesT = bases[l].transpose(1, 0, 2).reshape(_D, _R * _D)
        cmat = jnp.kron(comp[l].T, eye)
        biasl = bias[l].reshape(1, _D)
        if l == 0:
            hw, rp = _tc_proj(h, basesT, cmat, root[l], biasl)
        else:
            hw, rp = _tc_fused(p0, p1, rp, basesT, cmat, root[l], biasl)
        part = _sc_agg(hw.reshape(_NR, _D), edata, zrows)
        p0 = part[:_N]
        p1 = part[_N:]
    return _tc_final(p0, p1, rp)


# speedup vs baseline: 1.3443x; 1.3443x over previous
import functools

import jax
import jax.numpy as jnp
from jax import lax
from jax.experimental import pallas as pl
from jax.experimental.pallas import tpu as pltpu
from jax.experimental.pallas import tpu_sc as plsc

_N = 50000
_E = 1600000
_R = 8
_D = 32

_NC = 2
_NS = 16
_NW = _NC * _NS
_EW = _E // _NW
_K = 80
_CPS = 25
_SBE = _K * _CPS
_NSB = _EW // _SBE
_RPT = _N // _NS
_NR = _N * _R
_RPTC = _NR // _NS
_CROWS = _EW // _K
_NCH = _EW // _K

_BN = 2000


def _mesh():
    return plsc.VectorSubcoreMesh(core_axis_name="c", subcore_axis_name="s")


def _cnt_body(seg2_hbm, zc_hbm, out_hbm, seg_v, ones_v, acc_sh):
    c = lax.axis_index("c")
    s = lax.axis_index("s")
    wid = c * _NS + s
    pltpu.sync_copy(zc_hbm, acc_sh.at[pl.ds(s * _RPTC, _RPTC)])
    for t in range(_K // 16):
        ones_v[pl.ds(t * 16, 16)] = jnp.ones((16,), jnp.float32)
    plsc.subcore_barrier()
    cbase = wid * _CROWS

    def sb_body(j, carry):
        pltpu.sync_copy(seg2_hbm.at[pl.ds(cbase + j * _CPS, _CPS)], seg_v)

        def ch_body(i, c2):
            pltpu.sync_copy(ones_v, acc_sh.at[seg_v.at[i]], add=True)
            return c2

        return lax.fori_loop(0, _CPS, ch_body, carry)

    lax.fori_loop(0, _NSB, sb_body, 0)
    plsc.subcore_barrier()
    pltpu.sync_copy(acc_sh.at[pl.ds(s * _RPTC, _RPTC)],
                    out_hbm.at[pl.ds(c * _NR + s * _RPTC, _RPTC)])


def _sc_counts(seg2, zc):
    f = pl.kernel(
        _cnt_body,
        out_type=jax.ShapeDtypeStruct((_NC * _NR,), jnp.float32),
        mesh=_mesh(),
        compiler_params=pltpu.CompilerParams(use_tc_tiling_on_sc=False, needs_layout_passes=False),
        scratch_types=[
            pltpu.VMEM((_CPS, _K), jnp.int32),
            pltpu.VMEM((_K,), jnp.float32),
            pltpu.VMEM_SHARED((_NR,), jnp.float32),
        ],
    )
    return f(seg2, zc)


def _scl_body(seg_hbm, invc_hbm, out_hbm, seg0, seg1, scl0, scl1, g0, g1):
    seg = [seg0, seg1]
    scl = [scl0, scl1]
    gsem = [g0, g1]
    c_ax = lax.axis_index("c")
    s_ax = lax.axis_index("s")
    wid = c_ax * _NS + s_ax
    ebase = wid * _EW

    def fire(ps, r):
        pltpu.async_copy(invc_hbm.at[seg[ps].at[pl.ds(r * _K, _K)]],
                         scl[ps].at[pl.ds(r * _K, _K)], gsem[ps])

    def drain(ps):
        pltpu.make_async_copy(invc_hbm.at[seg[ps].at[pl.ds(0, _K)]],
                              scl[ps].at[pl.ds(0, _K)], gsem[ps]).wait()

    def sb_loop(jj, carry):
        for u in range(2):
            j = jj * 2 + u
            pltpu.sync_copy(seg_hbm.at[pl.ds(ebase + j * _SBE, _SBE)], seg[u])
            fire(u, 0)

            def ch(i, c2):
                @pl.when(i + 1 < _CPS)
                def _():
                    fire(u, i + 1)

                drain(u)
                return c2

            lax.fori_loop(0, _CPS, ch, 0)
            pltpu.sync_copy(scl[u], out_hbm.at[pl.ds(ebase + j * _SBE, _SBE)])
        return carry

    lax.fori_loop(0, _NSB // 2, sb_loop, 0)
    jt = _NSB - 1
    pltpu.sync_copy(seg_hbm.at[pl.ds(ebase + jt * _SBE, _SBE)], seg[0])
    fire(0, 0)

    def cht(i, c2):
        @pl.when(i + 1 < _CPS)
        def _():
            fire(0, i + 1)

        drain(0)
        return c2

    lax.fori_loop(0, _CPS, cht, 0)
    pltpu.sync_copy(scl[0], out_hbm.at[pl.ds(ebase + jt * _SBE, _SBE)])


def _sc_scale(seg, invc):
    f = pl.kernel(
        _scl_body,
        out_type=jax.ShapeDtypeStruct((_E,), jnp.float32),
        mesh=_mesh(),
        compiler_params=pltpu.CompilerParams(use_tc_tiling_on_sc=False, needs_layout_passes=False),
        scratch_types=[
            pltpu.VMEM((_SBE,), jnp.int32),
            pltpu.VMEM((_SBE,), jnp.int32),
            pltpu.VMEM((_SBE,), jnp.float32),
            pltpu.VMEM((_SBE,), jnp.float32),
            pltpu.SemaphoreType.DMA,
            pltpu.SemaphoreType.DMA,
        ],
    )
    return f(seg, invc)


def _agg_body(hw_hbm, edata_hbm, zrows_hbm, out_hbm,
              edata_v, acc_sh, rows0, rows1, srows0, srows1,
              g0, g1, s0, s1):
    rows = [rows0, rows1]
    srows = [srows0, srows1]
    gsem = [g0, g1]
    ssem = [s0, s1]
    c_ax = lax.axis_index("c")
    s_ax = lax.axis_index("s")
    wid = c_ax * _NS + s_ax
    pltpu.sync_copy(zrows_hbm, acc_sh.at[pl.ds(s_ax * _RPT, _RPT)])
    plsc.subcore_barrier()
    cbase = wid * _CROWS

    def lin_load(sbi):
        pltpu.sync_copy(edata_hbm.at[pl.ds(cbase + sbi * _CPS, _CPS)],
                        edata_v.at[lax.rem(sbi, 2)])

    def fire_gather(ps, r, q):
        pltpu.async_copy(hw_hbm.at[edata_v.at[ps, r, 0]], rows[q], gsem[q])

    def drain_gather(q):
        pltpu.make_async_copy(hw_hbm.at[edata_v.at[0, 0, 0]],
                              rows[q], gsem[q]).wait()

    def fire_scatter(ps, r, q):
        pltpu.async_copy(srows[q], acc_sh.at[edata_v.at[ps, r, 1]],
                         ssem[q], add=True)

    def drain_scatter(q):
        pltpu.make_async_copy(srows[q], acc_sh.at[edata_v.at[0, 0, 1]],
                              ssem[q]).wait()

    def compute(ps, r, q):
        psv = jnp.full((16,), ps, jnp.int32)
        rv = jnp.full((16,), r, jnp.int32)
        two = jnp.full((16,), 2, jnp.int32)
        bufq = rows[q]
        sbufq = srows[q]
        for k in range(_K):
            kv = jnp.full((16,), k, jnp.int32)
            sc = plsc.bitcast(plsc.load_gather(edata_v, [psv, rv, two, kv]),
                              jnp.float32)
            sbufq[k, pl.ds(0, 16)] = bufq[k, pl.ds(0, 16)] * sc
            sbufq[k, pl.ds(16, 16)] = bufq[k, pl.ds(16, 16)] * sc

    def posn(cc):
        sb = lax.div(cc, _CPS)
        return lax.rem(sb, 2), lax.rem(cc, _CPS)

    def chunk_step(c, q, last):
        cn = c + 1

        @pl.when(lax.rem(cn, _CPS) == 0)
        def _():
            lin_load(lax.div(cn, _CPS))

        psn, rn = posn(cn)
        fire_gather(psn, rn, 1 - q)
        drain_gather(q)

        @pl.when(c >= 2)
        def _():
            drain_scatter(q)

        ps, r = posn(c)
        compute(ps, r, q)
        fire_scatter(ps, r, q)

    lin_load(0)
    fire_gather(0, 0, 0)

    def step(t, carry):
        chunk_step(t * 2, 0, False)
        chunk_step(t * 2 + 1, 1, False)
        return carry

    lax.fori_loop(0, (_NCH - 1) // 2, step, 0)
    drain_gather(0)
    drain_scatter(0)
    ps_t, r_t = posn(_NCH - 1)
    compute(ps_t, r_t, 0)
    fire_scatter(ps_t, r_t, 0)
    drain_scatter(1)
    drain_scatter(0)
    plsc.subcore_barrier()
    pltpu.sync_copy(acc_sh.at[pl.ds(s_ax * _RPT, _RPT)],
                    out_hbm.at[pl.ds(c_ax * _N + s_ax * _RPT, _RPT)])


def _sc_agg(hw, edata, zrows):
    f = pl.kernel(
        _agg_body,
        out_type=jax.ShapeDtypeStruct((_NC * _N, _D), jnp.float32),
        mesh=_mesh(),
        compiler_params=pltpu.CompilerParams(use_tc_tiling_on_sc=False, needs_layout_passes=False),
        scratch_types=[
            pltpu.VMEM((2, _CPS, 3, _K), jnp.int32),
            pltpu.VMEM_SHARED((_N, _D), jnp.float32),
        ] + [pltpu.VMEM((_K, _D), jnp.float32)] * 4
          + [pltpu.SemaphoreType.DMA] * 4,
    )
    return f(hw, edata, zrows)


def _proj_body(h_ref, basesT_ref, cmat_ref, root_ref, bias_ref, cnt_ref,
               hw_ref, rp_ref, invc_ref):
    @pl.when(pl.program_id(0) == 0)
    def _():
        invc_ref[...] = 1.0 / jnp.maximum(cnt_ref[0] + cnt_ref[1], 1.0)

    h = h_ref[...]
    wcat = jnp.dot(basesT_ref[...], cmat_ref[...],
                   preferred_element_type=jnp.float32)
    hw_ref[...] = jnp.dot(h, wcat, preferred_element_type=jnp.float32)
    rp_ref[...] = (jnp.dot(h, root_ref[...], preferred_element_type=jnp.float32)
                   + bias_ref[...])


def _fused_body(p0_ref, p1_ref, rpp_ref, basesT_ref, cmat_ref, root_ref,
                bias_ref, hw_ref, rp_ref):
    h = jnp.maximum(p0_ref[...] + p1_ref[...] + rpp_ref[...], 0.0)
    wcat = jnp.dot(basesT_ref[...], cmat_ref[...],
                   preferred_element_type=jnp.float32)
    hw_ref[...] = jnp.dot(h, wcat, preferred_element_type=jnp.float32)
    rp_ref[...] = (jnp.dot(h, root_ref[...], preferred_element_type=jnp.float32)
                   + bias_ref[...])


_W_SPECS = [
    pl.BlockSpec((_D, _R * _D), lambda i: (0, 0)),
    pl.BlockSpec((_R * _D, _R * _D), lambda i: (0, 0)),
    pl.BlockSpec((_D, _D), lambda i: (0, 0)),
    pl.BlockSpec((1, _D), lambda i: (0, 0)),
]

_OUT_SPECS = [
    pl.BlockSpec((_BN, _R * _D), lambda i: (i, 0)),
    pl.BlockSpec((_BN, _D), lambda i: (i, 0)),
]

_OUT_SHAPES = [
    jax.ShapeDtypeStruct((_N, _R * _D), jnp.float32),
    jax.ShapeDtypeStruct((_N, _D), jnp.float32),
]


def _tc_proj(h, basesT, cmat, rootl, biasl, counts3):
    return pl.pallas_call(
        _proj_body,
        grid=(_N // _BN,),
        in_specs=[pl.BlockSpec((_BN, _D), lambda i: (i, 0))] + _W_SPECS
                 + [pl.BlockSpec((2, _NR // 128, 128), lambda i: (0, 0, 0))],
        out_specs=_OUT_SPECS + [pl.BlockSpec((_NR // 128, 128), lambda i: (0, 0))],
        out_shape=_OUT_SHAPES + [jax.ShapeDtypeStruct((_NR // 128, 128), jnp.float32)],
    )(h, basesT, cmat, rootl, biasl, counts3)


def _tc_fused(p0, p1, rpp, basesT, cmat, rootl, biasl):
    nspec = pl.BlockSpec((_BN, _D), lambda i: (i, 0))
    return pl.pallas_call(
        _fused_body,
        grid=(_N // _BN,),
        in_specs=[nspec, nspec, nspec] + _W_SPECS,
        out_specs=_OUT_SPECS,
        out_shape=_OUT_SHAPES,
    )(p0, p1, rpp, basesT, cmat, rootl, biasl)


def _fin_body(p0_ref, p1_ref, rp_ref, o_ref):
    o_ref[...] = p0_ref[...] + p1_ref[...] + rp_ref[...]


def _tc_final(p0, p1, rp):
    spec = pl.BlockSpec((2000, _D), lambda i: (i, 0))
    return pl.pallas_call(
        _fin_body,
        grid=(_N // 2000,),
        in_specs=[spec, spec, spec],
        out_specs=spec,
        out_shape=jax.ShapeDtypeStruct((_N, _D), jnp.float32),
    )(p0, p1, rp)


def kernel(x, edge_index, edge_type, embed, comp, bases, root, bias):
    h = jnp.take(embed, x, axis=0)
    src = edge_index[0]
    dst = edge_index[1]
    et = edge_type
    gidx = src * _R + et
    seg = dst * _R + et
    dst2 = dst.reshape(_E // _K, _K)
    seg2 = seg.reshape(_E // _K, _K)
    zrows = jnp.zeros((_RPT, _D), jnp.float32)
    zc = jnp.zeros((_RPTC,), jnp.float32)
    eye = jnp.eye(_D, dtype=jnp.float32)

    cnts = _sc_counts(seg2, zc)

    p0 = p1 = rp = None
    for l in range(3):
        basesT = bases[l].transpose(1, 0, 2).reshape(_D, _R * _D)
        cmat = jnp.kron(comp[l].T, eye)
        biasl = bias[l].reshape(1, _D)
        if l == 0:
            hw, rp, invc3 = _tc_proj(h, basesT, cmat, root[l], biasl,
                                     cnts.reshape(2, _NR // 128, 128))
            scale = _sc_scale(seg, invc3.reshape(_NR))
            scale_i = lax.bitcast_convert_type(scale, jnp.int32)
            edata = jnp.stack([gidx.reshape(_E // _K, _K), dst2,
                               scale_i.reshape(_E // _K, _K)], axis=1)
        else:
            hw, rp = _tc_fused(p0, p1, rp, basesT, cmat, root[l], biasl)
        part = _sc_agg(hw.reshape(_NR, _D), edata, zrows)
        p0 = part[:_N]
        p1 = part[_N:]
    return _tc_final(p0, p1, rp)


# speedup vs baseline: 1.3622x; 1.0133x over previous
import functools

import jax
import jax.numpy as jnp
from jax import lax
from jax.experimental import pallas as pl
from jax.experimental.pallas import tpu as pltpu
from jax.experimental.pallas import tpu_sc as plsc

_N = 50000
_E = 1600000
_R = 8
_D = 32

_NC = 2
_NS = 16
_NW = _NC * _NS
_EW = _E // _NW
_K = 80
_CPS = 25
_SBE = _K * _CPS
_NSB = _EW // _SBE
_RPT = _N // _NS
_NR = _N * _R
_RPTC = _NR // _NS
_CROWS = _EW // _K
_NCH = _EW // _K

_BN = 2000


def _mesh():
    return plsc.VectorSubcoreMesh(core_axis_name="c", subcore_axis_name="s")


def _cnt_body(seg2_hbm, zc_hbm, out_hbm, seg_v, ones_v, acc_sh):
    c = lax.axis_index("c")
    s = lax.axis_index("s")
    wid = c * _NS + s
    pltpu.sync_copy(zc_hbm, acc_sh.at[pl.ds(s * _RPTC, _RPTC)])
    for t in range(_K // 16):
        ones_v[pl.ds(t * 16, 16)] = jnp.ones((16,), jnp.float32)
    plsc.subcore_barrier()
    cbase = wid * _CROWS

    def sb_body(j, carry):
        pltpu.sync_copy(seg2_hbm.at[pl.ds(cbase + j * _CPS, _CPS)], seg_v)

        def ch_body(i, c2):
            pltpu.sync_copy(ones_v, acc_sh.at[seg_v.at[i]], add=True)
            return c2

        return lax.fori_loop(0, _CPS, ch_body, carry)

    lax.fori_loop(0, _NSB, sb_body, 0)
    plsc.subcore_barrier()
    pltpu.sync_copy(acc_sh.at[pl.ds(s * _RPTC, _RPTC)],
                    out_hbm.at[pl.ds(c * _NR + s * _RPTC, _RPTC)])


def _sc_counts(seg2, zc):
    f = pl.kernel(
        _cnt_body,
        out_type=jax.ShapeDtypeStruct((_NC * _NR,), jnp.float32),
        mesh=_mesh(),
        compiler_params=pltpu.CompilerParams(use_tc_tiling_on_sc=False, needs_layout_passes=False),
        scratch_types=[
            pltpu.VMEM((_CPS, _K), jnp.int32),
            pltpu.VMEM((_K,), jnp.float32),
            pltpu.VMEM_SHARED((_NR,), jnp.float32),
        ],
    )
    return f(seg2, zc)


def _scl_body(seg_hbm, invc_hbm, out_hbm, seg0, seg1, scl0, scl1, g0, g1):
    seg = [seg0, seg1]
    scl = [scl0, scl1]
    gsem = [g0, g1]
    c_ax = lax.axis_index("c")
    s_ax = lax.axis_index("s")
    wid = c_ax * _NS + s_ax
    ebase = wid * _EW

    def fire(ps, r):
        pltpu.async_copy(invc_hbm.at[seg[ps].at[pl.ds(r * _K, _K)]],
                         scl[ps].at[pl.ds(r * _K, _K)], gsem[ps])

    def drain(ps):
        pltpu.make_async_copy(invc_hbm.at[seg[ps].at[pl.ds(0, _K)]],
                              scl[ps].at[pl.ds(0, _K)], gsem[ps]).wait()

    def sb_loop(jj, carry):
        for u in range(2):
            j = jj * 2 + u
            pltpu.sync_copy(seg_hbm.at[pl.ds(ebase + j * _SBE, _SBE)], seg[u])
            fire(u, 0)

            def ch(i, c2):
                @pl.when(i + 1 < _CPS)
                def _():
                    fire(u, i + 1)

                drain(u)
                return c2

            lax.fori_loop(0, _CPS, ch, 0)
            pltpu.sync_copy(scl[u], out_hbm.at[pl.ds(ebase + j * _SBE, _SBE)])
        return carry

    lax.fori_loop(0, _NSB // 2, sb_loop, 0)
    jt = _NSB - 1
    pltpu.sync_copy(seg_hbm.at[pl.ds(ebase + jt * _SBE, _SBE)], seg[0])
    fire(0, 0)

    def cht(i, c2):
        @pl.when(i + 1 < _CPS)
        def _():
            fire(0, i + 1)

        drain(0)
        return c2

    lax.fori_loop(0, _CPS, cht, 0)
    pltpu.sync_copy(scl[0], out_hbm.at[pl.ds(ebase + jt * _SBE, _SBE)])


def _sc_scale(seg, invc):
    f = pl.kernel(
        _scl_body,
        out_type=jax.ShapeDtypeStruct((_E,), jnp.float32),
        mesh=_mesh(),
        compiler_params=pltpu.CompilerParams(use_tc_tiling_on_sc=False, needs_layout_passes=False),
        scratch_types=[
            pltpu.VMEM((_SBE,), jnp.int32),
            pltpu.VMEM((_SBE,), jnp.int32),
            pltpu.VMEM((_SBE,), jnp.float32),
            pltpu.VMEM((_SBE,), jnp.float32),
            pltpu.SemaphoreType.DMA,
            pltpu.SemaphoreType.DMA,
        ],
    )
    return f(seg, invc)


def _agg_body(hw_hbm, edata_hbm, zrows_hbm, out_hbm,
              edata_v, acc_sh, rows0, rows1, srows0, srows1,
              g0, g1, s0, s1):
    rows = [rows0, rows1]
    srows = [srows0, srows1]
    gsem = [g0, g1]
    ssem = [s0, s1]
    c_ax = lax.axis_index("c")
    s_ax = lax.axis_index("s")
    wid = c_ax * _NS + s_ax
    pltpu.sync_copy(zrows_hbm, acc_sh.at[pl.ds(s_ax * _RPT, _RPT)])
    plsc.subcore_barrier()
    cbase = wid * _CROWS

    def lin_load(sbi):
        pltpu.sync_copy(edata_hbm.at[pl.ds(cbase + sbi * _CPS, _CPS)],
                        edata_v.at[lax.rem(sbi, 2)])

    def fire_gather(ps, r, q):
        pltpu.async_copy(hw_hbm.at[edata_v.at[ps, r, 0]], rows[q], gsem[q])

    def drain_gather(q):
        pltpu.make_async_copy(hw_hbm.at[edata_v.at[0, 0, 0]],
                              rows[q], gsem[q]).wait()

    def fire_scatter(ps, r, q):
        pltpu.async_copy(srows[q], acc_sh.at[edata_v.at[ps, r, 1]],
                         ssem[q], add=True)

    def drain_scatter(q):
        pltpu.make_async_copy(srows[q], acc_sh.at[edata_v.at[0, 0, 1]],
                              ssem[q]).wait()

    def compute(ps, r, q):
        psv = jnp.full((16,), ps, jnp.int32)
        rv = jnp.full((16,), r, jnp.int32)
        two = jnp.full((16,), 2, jnp.int32)
        bufq = rows[q]
        sbufq = srows[q]
        for k in range(_K):
            kv = jnp.full((16,), k, jnp.int32)
            sc = plsc.bitcast(plsc.load_gather(edata_v, [psv, rv, two, kv]),
                              jnp.float32)
            sbufq[k, pl.ds(0, 16)] = bufq[k, pl.ds(0, 16)] * sc
            sbufq[k, pl.ds(16, 16)] = bufq[k, pl.ds(16, 16)] * sc

    def posn(cc):
        sb = lax.div(cc, _CPS)
        return lax.rem(sb, 2), lax.rem(cc, _CPS)

    def chunk_step(c, q, last):
        cn = c + 1

        @pl.when(lax.rem(cn, _CPS) == 0)
        def _():
            lin_load(lax.div(cn, _CPS))

        psn, rn = posn(cn)
        fire_gather(psn, rn, 1 - q)
        drain_gather(q)

        @pl.when(c >= 2)
        def _():
            drain_scatter(q)

        ps, r = posn(c)
        compute(ps, r, q)
        fire_scatter(ps, r, q)

    lin_load(0)
    fire_gather(0, 0, 0)

    def step(t, carry):
        chunk_step(t * 2, 0, False)
        chunk_step(t * 2 + 1, 1, False)
        return carry

    lax.fori_loop(0, (_NCH - 1) // 2, step, 0)
    drain_gather(0)
    drain_scatter(0)
    ps_t, r_t = posn(_NCH - 1)
    compute(ps_t, r_t, 0)
    fire_scatter(ps_t, r_t, 0)
    drain_scatter(1)
    drain_scatter(0)
    plsc.subcore_barrier()
    pltpu.sync_copy(acc_sh.at[pl.ds(s_ax * _RPT, _RPT)],
                    out_hbm.at[pl.ds(c_ax * _N + s_ax * _RPT, _RPT)])


def _sc_agg(hw, edata, zrows):
    f = pl.kernel(
        _agg_body,
        out_type=jax.ShapeDtypeStruct((_NC * _N, _D), jnp.float32),
        mesh=_mesh(),
        compiler_params=pltpu.CompilerParams(use_tc_tiling_on_sc=False, needs_layout_passes=False),
        scratch_types=[
            pltpu.VMEM((2, _CPS, 3, _K), jnp.int32),
            pltpu.VMEM_SHARED((_N, _D), jnp.float32),
        ] + [pltpu.VMEM((_K, _D), jnp.float32)] * 4
          + [pltpu.SemaphoreType.DMA] * 4,
    )
    return f(hw, edata, zrows)


def _invc_body(c_ref, o_ref):
    csum = c_ref[0] + c_ref[1]
    o_ref[...] = 1.0 / jnp.maximum(csum, 1.0)


def _tc_invc(counts3):
    return pl.pallas_call(
        _invc_body,
        grid=(1,),
        in_specs=[pl.BlockSpec((2, _NR // 128, 128), lambda i: (0, 0, 0))],
        out_specs=pl.BlockSpec((_NR // 128, 128), lambda i: (0, 0)),
        out_shape=jax.ShapeDtypeStruct((_NR // 128, 128), jnp.float32),
    )(counts3)


def _proj_body(h_ref, basesT_ref, cmat_ref, root_ref, bias_ref, hw_ref, rp_ref):
    h = h_ref[...]
    wcat = jnp.dot(basesT_ref[...], cmat_ref[...],
                   preferred_element_type=jnp.float32)
    hw_ref[...] = jnp.dot(h, wcat, preferred_element_type=jnp.float32)
    rp_ref[...] = (jnp.dot(h, root_ref[...], preferred_element_type=jnp.float32)
                   + bias_ref[...])


def _fused_body(p0_ref, p1_ref, rpp_ref, basesT_ref, cmat_ref, root_ref,
                bias_ref, hw_ref, rp_ref):
    h = jnp.maximum(p0_ref[...] + p1_ref[...] + rpp_ref[...], 0.0)
    wcat = jnp.dot(basesT_ref[...], cmat_ref[...],
                   preferred_element_type=jnp.float32)
    hw_ref[...] = jnp.dot(h, wcat, preferred_element_type=jnp.float32)
    rp_ref[...] = (jnp.dot(h, root_ref[...], preferred_element_type=jnp.float32)
                   + bias_ref[...])


_W_SPECS = [
    pl.BlockSpec((_D, _R * _D), lambda i: (0, 0)),
    pl.BlockSpec((_R * _D, _R * _D), lambda i: (0, 0)),
    pl.BlockSpec((_D, _D), lambda i: (0, 0)),
    pl.BlockSpec((1, _D), lambda i: (0, 0)),
]

_OUT_SPECS = [
    pl.BlockSpec((_BN, _R * _D), lambda i: (i, 0)),
    pl.BlockSpec((_BN, _D), lambda i: (i, 0)),
]

_OUT_SHAPES = [
    jax.ShapeDtypeStruct((_N, _R * _D), jnp.float32),
    jax.ShapeDtypeStruct((_N, _D), jnp.float32),
]


def _tc_proj(h, basesT, cmat, rootl, biasl):
    return pl.pallas_call(
        _proj_body,
        grid=(_N // _BN,),
        in_specs=[pl.BlockSpec((_BN, _D), lambda i: (i, 0))] + _W_SPECS,
        out_specs=_OUT_SPECS,
        out_shape=_OUT_SHAPES,
    )(h, basesT, cmat, rootl, biasl)


def _tc_fused(p0, p1, rpp, basesT, cmat, rootl, biasl):
    nspec = pl.BlockSpec((_BN, _D), lambda i: (i, 0))
    return pl.pallas_call(
        _fused_body,
        grid=(_N // _BN,),
        in_specs=[nspec, nspec, nspec] + _W_SPECS,
        out_specs=_OUT_SPECS,
        out_shape=_OUT_SHAPES,
    )(p0, p1, rpp, basesT, cmat, rootl, biasl)


def _fin_body(p0_ref, p1_ref, rp_ref, o_ref):
    o_ref[...] = p0_ref[...] + p1_ref[...] + rp_ref[...]


def _tc_final(p0, p1, rp):
    spec = pl.BlockSpec((2000, _D), lambda i: (i, 0))
    return pl.pallas_call(
        _fin_body,
        grid=(_N // 2000,),
        in_specs=[spec, spec, spec],
        out_specs=spec,
        out_shape=jax.ShapeDtypeStruct((_N, _D), jnp.float32),
    )(p0, p1, rp)


def kernel(x, edge_index, edge_type, embed, comp, bases, root, bias):
    h = jnp.take(embed, x, axis=0)
    src = edge_index[0]
    dst = edge_index[1]
    et = edge_type
    gidx = src * _R + et
    seg = dst * _R + et
    dst2 = dst.reshape(_E // _K, _K)
    seg2 = seg.reshape(_E // _K, _K)
    zrows = jnp.zeros((_RPT, _D), jnp.float32)
    zc = jnp.zeros((_RPTC,), jnp.float32)
    eye = jnp.eye(_D, dtype=jnp.float32)

    cnts = _sc_counts(seg2, zc)
    invc = _tc_invc(cnts.reshape(2, _NR // 128, 128)).reshape(_NR)
    scale = _sc_scale(seg, invc)
    scale_i = lax.bitcast_convert_type(scale, jnp.int32)
    edata = jnp.stack([gidx.reshape(_E // _K, _K), dst2,
                       scale_i.reshape(_E // _K, _K)], axis=1)

    p0 = p1 = rp = None
    for l in range(3):
        basesT = bases[l].transpose(1, 0, 2).reshape(_D, _R * _D)
        cmat = jnp.kron(comp[l].T, eye)
        biasl = bias[l].reshape(1, _D)
        if l == 0:
            hw, rp = _tc_proj(h, basesT, cmat, root[l], biasl)
        else:
            hw, rp = _tc_fused(p0, p1, rp, basesT, cmat, root[l], biasl)
        part = _sc_agg(hw.reshape(_NR, _D), edata, zrows)
        p0 = part[:_N]
        p1 = part[_N:]
    return _tc_final(p0, p1, rp)
